# baseline (speedup 1.0000x reference)
import jax
import jax.numpy as jnp
from jax.experimental import pallas as pl
from jax.experimental.pallas import tpu as pltpu

TOPK = 154
NE = 512
BM = 256
C21 = float(1 << 21)
SIGN32 = -2147483648


def _lane_iota():
    return jax.lax.broadcasted_iota(jnp.int32, (1, 128), 1)


def _perm_mat(j):
    a = jax.lax.broadcasted_iota(jnp.int32, (128, 128), 0)
    b = jax.lax.broadcasted_iota(jnp.int32, (128, 128), 1)
    return ((a ^ j) == b).astype(jnp.float32)


def _self_greater(da, db):
    return da * C21 + db > 0.0


def _cx_within(cols, j, masks):
    pmat = _perm_mat(j)
    out = []
    for (a, bw), tm in zip(cols, masks):
        pa = jnp.dot(a, pmat, preferred_element_type=jnp.float32)
        pb = jnp.dot(bw, pmat, preferred_element_type=jnp.float32)
        take_self = _self_greater(a - pa, bw - pb) == tm
        out.append((jnp.where(take_self, a, pa),
                    jnp.where(take_self, bw, pb)))
    return out


def _cx_cross(cols, jc, dirs):
    out = list(cols)
    for c in range(len(cols)):
        p = c ^ jc
        if p <= c or p >= len(cols):
            continue
        (aa, ab), (ba, bb) = cols[c], cols[p]
        gt = _self_greater(aa - ba, ab - bb)
        hi = (jnp.where(gt, aa, ba), jnp.where(gt, ab, bb))
        lo = (jnp.where(gt, ba, aa), jnp.where(gt, bb, ab))
        out[c], out[p] = (hi, lo) if dirs[c] else (lo, hi)
    return out


def _topk_sort(cols):
    lane = _lane_iota()

    for kp in range(1, 7):
        k = 1 << kp
        j = k >> 1
        while j >= 1:
            tm = ((lane & k) == 0) == ((lane & j) == 0)
            cols = _cx_within(cols, j, [tm] * 4)
            j >>= 1

    for jp in range(6, -1, -1):
        j = 1 << jp
        m_desc = (lane & j) == 0
        m_asc = jnp.logical_not(m_desc)
        cols = _cx_within(cols, j, [m_desc, m_asc, m_desc, m_asc])

    cols = _cx_cross(cols, 1, [True, True, False, False])
    for jp in range(6, -1, -1):
        j = 1 << jp
        m_desc = (lane & j) == 0
        m_asc = jnp.logical_not(m_desc)
        cols = _cx_within(cols, j, [m_desc, m_desc, m_asc, m_asc])

    cols = _cx_cross(cols, 2, [True] * 4)
    cols = cols[:2]
    cols = _cx_cross(cols, 1, [True, True])
    for jp in range(6, -1, -1):
        j = 1 << jp
        m_desc = (lane & j) == 0
        cols = _cx_within(cols, j, [m_desc, m_desc])
    return cols


def _fused_body(x_ref, wt_ref, b_ref, w_ref, i_ref):
    logits = (
        jnp.dot(x_ref[...], wt_ref[...], preferred_element_type=jnp.float32)
        + b_ref[...]
    )

    bits = jax.lax.bitcast_convert_type(logits, jnp.int32)
    s = bits ^ ((bits >> 31) | SIGN32)
    a_all = jax.lax.shift_right_logical(s, 11)
    blo_all = (s & 0x7FF) * 512

    lane = _lane_iota()
    cols = []
    for c in range(NE // 128):
        sl = slice(c * 128, (c + 1) * 128)
        comp_idx = 511 - (lane + c * 128)
        a = a_all[:, sl].astype(jnp.float32)
        bw = (blo_all[:, sl] + comp_idx).astype(jnp.float32)
        cols.append((a, bw))

    top = _topk_sort(cols)
    a_t = jnp.concatenate([t[0] for t in top], axis=1)[:, :TOPK]
    b_t = jnp.concatenate([t[1] for t in top], axis=1)[:, :TOPK]

    a_i = a_t.astype(jnp.int32)
    b_i = b_t.astype(jnp.int32)
    s_t = (a_i << 11) | jax.lax.shift_right_logical(b_i, 9)
    vbits = s_t ^ ((jnp.bitwise_not(s_t) >> 31) | SIGN32)
    vals = jax.lax.bitcast_convert_type(vbits, jnp.float32)
    idx = 511 - (b_i & 511)

    e = jnp.exp(vals - vals[:, 0:1])
    w = e / jnp.sum(e, axis=1, keepdims=True)
    w_ref[...] = w
    i_ref[...] = idx


def kernel(x, W, b):
    B, D = x.shape
    assert W.shape[0] == NE and B % BM == 0
    wt = W.T
    b2 = b.reshape(1, NE)
    weights, indices = pl.pallas_call(
        _fused_body,
        grid=(B // BM,),
        in_specs=[
            pl.BlockSpec((BM, D), lambda i: (i, 0)),
            pl.BlockSpec((D, NE), lambda i: (0, 0)),
            pl.BlockSpec((1, NE), lambda i: (0, 0)),
        ],
        out_specs=[
            pl.BlockSpec((BM, TOPK), lambda i: (i, 0)),
            pl.BlockSpec((BM, TOPK), lambda i: (i, 0)),
        ],
        out_shape=[
            jax.ShapeDtypeStruct((B, TOPK), jnp.float32),
            jax.ShapeDtypeStruct((B, TOPK), jnp.int32),
        ],
        compiler_params=pltpu.CompilerParams(
            dimension_semantics=("parallel",),
        ),
    )(x, wt, b2)
    return (weights, indices)
